# Initial kernel scaffold; baseline (speedup 1.0000x reference)
import jax
import jax.numpy as jnp
from jax.experimental import pallas as pl


def kernel(x, style, mod1_w, mod1_b, w1, act1_b, mod2_w, mod2_b, w2, act2_b, skip_w, skip_act_b):
    raise NotImplementedError("write your pallas kernel here")


# speedup vs baseline: 10.3135x; 10.3135x over previous
import functools
import math

import numpy as np

import jax
import jax.numpy as jnp
from jax.experimental import pallas as pl
from jax.experimental.pallas import tpu as pltpu

_SQRT2 = math.sqrt(2.0)
_INV_SQRT2 = 1.0 / math.sqrt(2.0)


def _lrelu_scaled(v):
    return jnp.where(v >= 0.0, v, 0.2 * v) * _SQRT2


def _phase_row_ops():
    kh = np.array([1.0, 3.0, 3.0, 1.0], np.float64) / 4.0
    r = np.zeros((2, 3, 3), np.float64)
    for q in range(2):
        for ky in range(3):
            for a in range(3):
                idx = 2 * a + ky - 1 - q
                if 0 <= idx < 4:
                    r[q, ky, a] = kh[idx]
    return r.astype(np.float32)


def _fused_block_kernel(x_ref, s1_ref, s2t_ref, w1s_ref, b1s_ref,
                        wsk_ref, bsk_ref, w2f_ref, b2_ref, o_ref,
                        *, h, w, cin, cout):
    n = h * w
    x = x_ref[0]
    pos = jax.lax.broadcasted_iota(jnp.int32, (1, n), 1)
    colv = pos % w
    rowv = pos // w

    def shift2d(v, dy, dx):
        s = dy * w + dx
        if s == 0:
            sh = v
        elif s > 0:
            sh = jnp.concatenate(
                [v[:, s:], jnp.zeros((v.shape[0], s), v.dtype)], axis=1)
        else:
            sh = jnp.concatenate(
                [jnp.zeros((v.shape[0], -s), v.dtype), v[:, :n + s]], axis=1)
        if dx != 0:
            sh = jnp.where((colv + dx >= 0) & (colv + dx < w), sh, 0.0)
        return sh

    xm = x * s1_ref[0]
    xs = jnp.concatenate(
        [shift2d(xm, dy, dx) for dy in (-1, 0, 1) for dx in (-1, 0, 1)],
        axis=0)
    t = jnp.dot(w1s_ref[...], xs, preferred_element_type=jnp.float32)
    y1 = _lrelu_scaled(t + b1s_ref[...])
    ym = y1 * s2t_ref[0]

    z = _lrelu_scaled(
        jnp.dot(wsk_ref[...], x, preferred_element_type=jnp.float32)
        + bsk_ref[...])
    zup = shift2d(z, -1, 0) + jnp.where(rowv == 0, z, 0.0)
    zdn = shift2d(z, 1, 0) + jnp.where(rowv == h - 1, z, 0.0)
    r0 = 0.75 * z + 0.25 * zup
    r1 = 0.75 * z + 0.25 * zdn

    def colmix(v):
        vl = shift2d(v, 0, -1) + jnp.where(colv == 0, v, 0.0)
        vr = shift2d(v, 0, 1) + jnp.where(colv == w - 1, v, 0.0)
        return 0.75 * v + 0.25 * vl, 0.75 * v + 0.25 * vr

    s00, s01 = colmix(r0)
    s10, s11 = colmix(r1)
    skip_all = jnp.concatenate([s00, s01, s10, s11], axis=1)

    cache = {}

    def plane_shifted(py, px, sy, sx):
        key = (py, px, sy, sx)
        if key not in cache:
            q = py * 2 + px
            cache[key] = shift2d(ym[q * cout:(q + 1) * cout], sy, sx)
        return cache[key]

    rowblocks = []
    for dy in (-1, 0, 1):
        for dx in (-1, 0, 1):
            qparts = []
            for qy in (0, 1):
                for qx in (0, 1):
                    py, sy = (qy + dy) % 2, (qy + dy) // 2
                    px, sx = (qx + dx) % 2, (qx + dx) // 2
                    qparts.append(plane_shifted(py, px, sy, sx))
            rowblocks.append(jnp.concatenate(qparts, axis=1))
    xs2 = jnp.concatenate(rowblocks, axis=0)

    res = _lrelu_scaled(
        jnp.dot(w2f_ref[...], xs2, preferred_element_type=jnp.float32)
        + b2_ref[...])
    o_ref[0] = (skip_all + res) * _INV_SQRT2


def kernel(x, style, mod1_w, mod1_b, w1, act1_b,
           mod2_w, mod2_b, w2, act2_b, skip_w, skip_act_b):
    B, Cin, H, W = x.shape
    S = style.shape[1]
    Cout = w1.shape[1]
    N = H * W
    f32 = jnp.float32

    with jax.default_matmul_precision("highest"):
        def modulate(mw, mb):
            s = style @ (mw * (1.0 / math.sqrt(S))).T + mb
            return s * jax.lax.rsqrt(jnp.mean(s * s, axis=1, keepdims=True) + 1e-8)

        def demod_weight(w_raw, cin_):
            wd = w_raw[0] * (1.0 / math.sqrt(cin_ * 9))
            d = jax.lax.rsqrt(jnp.sum(wd * wd, axis=(1, 2, 3)) + 1e-8)
            return wd * d[:, None, None, None]

        s1 = modulate(mod1_w, mod1_b)
        s2 = modulate(mod2_w, mod2_b)
        w1d = demod_weight(w1, Cin)
        w2d = demod_weight(w2, Cout)

        rm = jnp.asarray(_phase_row_ops())
        w1s = jnp.einsum("ocyx,qya,rxb->qroabc", w1d, rm, rm,
                         ).reshape(4 * Cout, 9 * Cin)
        w2f = jnp.transpose(w2d, (0, 2, 3, 1)).reshape(Cout, 9 * Cout)

        s1col = s1[..., None]
        s2t = jnp.tile(s2, (1, 4))[..., None]
        b1s = jnp.tile(act1_b, 4)[:, None]
        wsk = skip_w * (1.0 / math.sqrt(Cin))
        bsk = skip_act_b[:, None]
        b2 = act2_b[:, None]

        x2 = x.reshape(B, Cin, N)

        outp = pl.pallas_call(
            functools.partial(_fused_block_kernel, h=H, w=W, cin=Cin, cout=Cout),
            out_shape=jax.ShapeDtypeStruct((B, Cout, 4 * N), f32),
            grid=(B,),
            in_specs=[
                pl.BlockSpec((1, Cin, N), lambda b: (b, 0, 0)),
                pl.BlockSpec((1, Cin, 1), lambda b: (b, 0, 0)),
                pl.BlockSpec((1, 4 * Cout, 1), lambda b: (b, 0, 0)),
                pl.BlockSpec((4 * Cout, 9 * Cin), lambda b: (0, 0)),
                pl.BlockSpec((4 * Cout, 1), lambda b: (0, 0)),
                pl.BlockSpec((Cout, Cin), lambda b: (0, 0)),
                pl.BlockSpec((Cout, 1), lambda b: (0, 0)),
                pl.BlockSpec((Cout, 9 * Cout), lambda b: (0, 0)),
                pl.BlockSpec((Cout, 1), lambda b: (0, 0)),
            ],
            out_specs=pl.BlockSpec((1, Cout, 4 * N), lambda b: (b, 0, 0)),
            compiler_params=pltpu.CompilerParams(
                dimension_semantics=("parallel",)),
        )(x2, s1col, s2t, w1s, b1s, wsk, bsk, w2f, b2)

        out = outp.reshape(B, Cout, 2, 2, H, W)
        out = out.transpose(0, 1, 4, 2, 5, 3).reshape(B, Cout, 2 * H, 2 * W)
        return out


# speedup vs baseline: 21.3628x; 2.0713x over previous
import functools
import math

import numpy as np

import jax
import jax.numpy as jnp
from jax.experimental import pallas as pl
from jax.experimental.pallas import tpu as pltpu

_SQRT2 = math.sqrt(2.0)
_INV_SQRT2 = 1.0 / math.sqrt(2.0)


def _lrelu_scaled(v):
    return jnp.where(v >= 0.0, v, 0.2 * v) * _SQRT2


def _phase_row_ops():
    kh = np.array([1.0, 3.0, 3.0, 1.0], np.float64) / 4.0
    r = np.zeros((2, 3, 3), np.float64)
    for q in range(2):
        for ky in range(3):
            for a in range(3):
                idx = 2 * a + ky - 1 - q
                if 0 <= idx < 4:
                    r[q, ky, a] = kh[idx]
    return r.astype(np.float32)


def _fused_block_kernel(x_ref, s1_ref, s2t_ref, w1s_ref, b1s_ref,
                        wsk_ref, bsk_ref, w2f_ref, b2_ref, o_ref,
                        *, h, w, cin, cout):
    n = h * w
    x = x_ref[0]
    pos = jax.lax.broadcasted_iota(jnp.int32, (1, n), 1)
    colv = pos % w
    rowv = pos // w

    def shift2d(v, dy, dx):
        s = dy * w + dx
        if s == 0:
            sh = v
        elif s > 0:
            sh = jnp.concatenate(
                [v[:, s:], jnp.zeros((v.shape[0], s), v.dtype)], axis=1)
        else:
            sh = jnp.concatenate(
                [jnp.zeros((v.shape[0], -s), v.dtype), v[:, :n + s]], axis=1)
        if dx != 0:
            sh = jnp.where((colv + dx >= 0) & (colv + dx < w), sh, 0.0)
        return sh

    xm = (x * s1_ref[0]).astype(jnp.bfloat16)
    xs = jnp.concatenate(
        [shift2d(xm, dy, dx) for dy in (-1, 0, 1) for dx in (-1, 0, 1)],
        axis=0)
    t = jnp.dot(w1s_ref[...], xs, preferred_element_type=jnp.float32,
                precision=jax.lax.Precision.DEFAULT)
    y1 = _lrelu_scaled(t + b1s_ref[...])
    ym = (y1 * s2t_ref[0]).astype(jnp.bfloat16)

    z = _lrelu_scaled(
        jnp.dot(wsk_ref[...], x, preferred_element_type=jnp.float32)
        + bsk_ref[...])
    zup = shift2d(z, -1, 0) + jnp.where(rowv == 0, z, 0.0)
    zdn = shift2d(z, 1, 0) + jnp.where(rowv == h - 1, z, 0.0)
    r0 = 0.75 * z + 0.25 * zup
    r1 = 0.75 * z + 0.25 * zdn

    def colmix(v):
        vl = shift2d(v, 0, -1) + jnp.where(colv == 0, v, 0.0)
        vr = shift2d(v, 0, 1) + jnp.where(colv == w - 1, v, 0.0)
        return 0.75 * v + 0.25 * vl, 0.75 * v + 0.25 * vr

    s00, s01 = colmix(r0)
    s10, s11 = colmix(r1)
    skip_all = jnp.concatenate([s00, s01, s10, s11], axis=1)

    cache = {}

    def plane_shifted(py, px, sy, sx):
        key = (py, px, sy, sx)
        if key not in cache:
            q = py * 2 + px
            cache[key] = shift2d(ym[q * cout:(q + 1) * cout], sy, sx)
        return cache[key]

    rowblocks = []
    for dy in (-1, 0, 1):
        for dx in (-1, 0, 1):
            qparts = []
            for qy in (0, 1):
                for qx in (0, 1):
                    py, sy = (qy + dy) % 2, (qy + dy) // 2
                    px, sx = (qx + dx) % 2, (qx + dx) // 2
                    qparts.append(plane_shifted(py, px, sy, sx))
            rowblocks.append(jnp.concatenate(qparts, axis=1))
    xs2 = jnp.concatenate(rowblocks, axis=0)

    res = _lrelu_scaled(
        jnp.dot(w2f_ref[...], xs2, preferred_element_type=jnp.float32,
                precision=jax.lax.Precision.DEFAULT)
        + b2_ref[...])
    o_ref[0] = (skip_all + res) * _INV_SQRT2


def kernel(x, style, mod1_w, mod1_b, w1, act1_b,
           mod2_w, mod2_b, w2, act2_b, skip_w, skip_act_b):
    B, Cin, H, W = x.shape
    S = style.shape[1]
    Cout = w1.shape[1]
    N = H * W
    f32 = jnp.float32

    with jax.default_matmul_precision("highest"):
        def modulate(mw, mb):
            s = style @ (mw * (1.0 / math.sqrt(S))).T + mb
            return s * jax.lax.rsqrt(jnp.mean(s * s, axis=1, keepdims=True) + 1e-8)

        def demod_weight(w_raw, cin_):
            wd = w_raw[0] * (1.0 / math.sqrt(cin_ * 9))
            d = jax.lax.rsqrt(jnp.sum(wd * wd, axis=(1, 2, 3)) + 1e-8)
            return wd * d[:, None, None, None]

        s1 = modulate(mod1_w, mod1_b)
        s2 = modulate(mod2_w, mod2_b)
        w1d = demod_weight(w1, Cin)
        w2d = demod_weight(w2, Cout)

        rm = jnp.asarray(_phase_row_ops())
        w1s = jnp.einsum("ocyx,qya,rxb->qroabc", w1d, rm, rm,
                         ).reshape(4 * Cout, 9 * Cin).astype(jnp.bfloat16)
        w2f = jnp.transpose(w2d, (0, 2, 3, 1)).reshape(
            Cout, 9 * Cout).astype(jnp.bfloat16)

        s1col = s1[..., None]
        s2t = jnp.tile(s2, (1, 4))[..., None]
        b1s = jnp.tile(act1_b, 4)[:, None]
        wsk = skip_w * (1.0 / math.sqrt(Cin))
        bsk = skip_act_b[:, None]
        b2 = act2_b[:, None]

        x2 = x.reshape(B, Cin, N)

        outp = pl.pallas_call(
            functools.partial(_fused_block_kernel, h=H, w=W, cin=Cin, cout=Cout),
            out_shape=jax.ShapeDtypeStruct((B, Cout, 4 * N), f32),
            grid=(B,),
            in_specs=[
                pl.BlockSpec((1, Cin, N), lambda b: (b, 0, 0)),
                pl.BlockSpec((1, Cin, 1), lambda b: (b, 0, 0)),
                pl.BlockSpec((1, 4 * Cout, 1), lambda b: (b, 0, 0)),
                pl.BlockSpec((4 * Cout, 9 * Cin), lambda b: (0, 0)),
                pl.BlockSpec((4 * Cout, 1), lambda b: (0, 0)),
                pl.BlockSpec((Cout, Cin), lambda b: (0, 0)),
                pl.BlockSpec((Cout, 1), lambda b: (0, 0)),
                pl.BlockSpec((Cout, 9 * Cout), lambda b: (0, 0)),
                pl.BlockSpec((Cout, 1), lambda b: (0, 0)),
            ],
            out_specs=pl.BlockSpec((1, Cout, 4 * N), lambda b: (b, 0, 0)),
            compiler_params=pltpu.CompilerParams(
                dimension_semantics=("parallel",)),
        )(x2, s1col, s2t, w1s, b1s, wsk, bsk, w2f, b2)

        out = outp.reshape(B, Cout, 2, 2, H, W)
        out = out.transpose(0, 1, 4, 2, 5, 3).reshape(B, Cout, 2 * H, 2 * W)
        return out


# speedup vs baseline: 22.4704x; 1.0519x over previous
import functools
import math

import numpy as np

import jax
import jax.numpy as jnp
from jax.experimental import pallas as pl
from jax.experimental.pallas import tpu as pltpu

_SQRT2 = math.sqrt(2.0)
_INV_SQRT2 = 1.0 / math.sqrt(2.0)


def _lrelu_scaled(v):
    return jnp.where(v >= 0.0, v, 0.2 * v) * _SQRT2


def _phase_row_ops():
    kh = np.array([1.0, 3.0, 3.0, 1.0], np.float64) / 4.0
    r = np.zeros((2, 3, 3), np.float64)
    for q in range(2):
        for ky in range(3):
            for a in range(3):
                idx = 2 * a + ky - 1 - q
                if 0 <= idx < 4:
                    r[q, ky, a] = kh[idx]
    return r.astype(np.float32)


def _phase_perm(h, w):
    n = h * w
    p = np.zeros((4 * n, 4 * n), np.float32)
    for qy in range(2):
        for qx in range(2):
            for y in range(h):
                for x in range(w):
                    src = (qy * 2 + qx) * n + y * w + x
                    dst = (2 * y + qy) * 2 * w + (2 * x + qx)
                    p[src, dst] = 1.0
    return p


def _fused_block_kernel(x_ref, s1_ref, s2t_ref, w1s_ref, b1s_ref,
                        wsk_ref, bsk_ref, w2f_ref, b2_ref, perm_ref, o_ref,
                        *, h, w, cin, cout):
    n = h * w
    x = x_ref[0]
    pos = jax.lax.broadcasted_iota(jnp.int32, (1, n), 1)
    colv = pos % w
    rowv = pos // w

    def shift2d(v, dy, dx):
        s = dy * w + dx
        if s == 0:
            sh = v
        elif s > 0:
            sh = jnp.concatenate(
                [v[:, s:], jnp.zeros((v.shape[0], s), v.dtype)], axis=1)
        else:
            sh = jnp.concatenate(
                [jnp.zeros((v.shape[0], -s), v.dtype), v[:, :n + s]], axis=1)
        if dx != 0:
            sh = jnp.where((colv + dx >= 0) & (colv + dx < w), sh, 0.0)
        return sh

    xm = (x * s1_ref[0]).astype(jnp.bfloat16)
    xs = jnp.concatenate(
        [shift2d(xm, dy, dx) for dy in (-1, 0, 1) for dx in (-1, 0, 1)],
        axis=0)
    t = jnp.dot(w1s_ref[...], xs, preferred_element_type=jnp.float32,
                precision=jax.lax.Precision.DEFAULT)
    y1 = _lrelu_scaled(t + b1s_ref[...])
    ym = (y1 * s2t_ref[0]).astype(jnp.bfloat16)

    z = _lrelu_scaled(
        jnp.dot(wsk_ref[...], x, preferred_element_type=jnp.float32)
        + bsk_ref[...])
    zup = shift2d(z, -1, 0) + jnp.where(rowv == 0, z, 0.0)
    zdn = shift2d(z, 1, 0) + jnp.where(rowv == h - 1, z, 0.0)
    r0 = 0.75 * z + 0.25 * zup
    r1 = 0.75 * z + 0.25 * zdn

    def colmix(v):
        vl = shift2d(v, 0, -1) + jnp.where(colv == 0, v, 0.0)
        vr = shift2d(v, 0, 1) + jnp.where(colv == w - 1, v, 0.0)
        return 0.75 * v + 0.25 * vl, 0.75 * v + 0.25 * vr

    s00, s01 = colmix(r0)
    s10, s11 = colmix(r1)
    skip_all = jnp.concatenate([s00, s01, s10, s11], axis=1)

    cache = {}

    def plane_shifted(py, px, sy, sx):
        key = (py, px, sy, sx)
        if key not in cache:
            q = py * 2 + px
            cache[key] = shift2d(ym[q * cout:(q + 1) * cout], sy, sx)
        return cache[key]

    rowblocks = []
    for dy in (-1, 0, 1):
        for dx in (-1, 0, 1):
            qparts = []
            for qy in (0, 1):
                for qx in (0, 1):
                    py, sy = (qy + dy) % 2, (qy + dy) // 2
                    px, sx = (qx + dx) % 2, (qx + dx) // 2
                    qparts.append(plane_shifted(py, px, sy, sx))
            rowblocks.append(jnp.concatenate(qparts, axis=1))
    xs2 = jnp.concatenate(rowblocks, axis=0)

    res = _lrelu_scaled(
        jnp.dot(w2f_ref[...], xs2, preferred_element_type=jnp.float32,
                precision=jax.lax.Precision.DEFAULT)
        + b2_ref[...])
    outsum = ((skip_all + res) * _INV_SQRT2).astype(jnp.bfloat16)
    o_ref[0] = jnp.dot(outsum, perm_ref[...],
                       preferred_element_type=jnp.float32,
                       precision=jax.lax.Precision.DEFAULT)


def kernel(x, style, mod1_w, mod1_b, w1, act1_b,
           mod2_w, mod2_b, w2, act2_b, skip_w, skip_act_b):
    B, Cin, H, W = x.shape
    S = style.shape[1]
    Cout = w1.shape[1]
    N = H * W
    f32 = jnp.float32

    with jax.default_matmul_precision("highest"):
        def modulate(mw, mb):
            s = style @ (mw * (1.0 / math.sqrt(S))).T + mb
            return s * jax.lax.rsqrt(jnp.mean(s * s, axis=1, keepdims=True) + 1e-8)

        def demod_weight(w_raw, cin_):
            wd = w_raw[0] * (1.0 / math.sqrt(cin_ * 9))
            d = jax.lax.rsqrt(jnp.sum(wd * wd, axis=(1, 2, 3)) + 1e-8)
            return wd * d[:, None, None, None]

        s1 = modulate(mod1_w, mod1_b)
        s2 = modulate(mod2_w, mod2_b)
        w1d = demod_weight(w1, Cin)
        w2d = demod_weight(w2, Cout)

        rm = jnp.asarray(_phase_row_ops())
        w1s = jnp.einsum("ocyx,qya,rxb->qroabc", w1d, rm, rm,
                         ).reshape(4 * Cout, 9 * Cin).astype(jnp.bfloat16)
        w2f = jnp.transpose(w2d, (0, 2, 3, 1)).reshape(
            Cout, 9 * Cout).astype(jnp.bfloat16)

        s1col = s1[..., None]
        s2t = jnp.tile(s2, (1, 4))[..., None]
        b1s = jnp.tile(act1_b, 4)[:, None]
        wsk = skip_w * (1.0 / math.sqrt(Cin))
        bsk = skip_act_b[:, None]
        b2 = act2_b[:, None]

        x2 = x.reshape(B, Cin, N)
        perm = jnp.asarray(_phase_perm(H, W), jnp.bfloat16)

        outp = pl.pallas_call(
            functools.partial(_fused_block_kernel, h=H, w=W, cin=Cin, cout=Cout),
            out_shape=jax.ShapeDtypeStruct((B, Cout, 4 * N), f32),
            grid=(B,),
            in_specs=[
                pl.BlockSpec((1, Cin, N), lambda b: (b, 0, 0)),
                pl.BlockSpec((1, Cin, 1), lambda b: (b, 0, 0)),
                pl.BlockSpec((1, 4 * Cout, 1), lambda b: (b, 0, 0)),
                pl.BlockSpec((4 * Cout, 9 * Cin), lambda b: (0, 0)),
                pl.BlockSpec((4 * Cout, 1), lambda b: (0, 0)),
                pl.BlockSpec((Cout, Cin), lambda b: (0, 0)),
                pl.BlockSpec((Cout, 1), lambda b: (0, 0)),
                pl.BlockSpec((Cout, 9 * Cout), lambda b: (0, 0)),
                pl.BlockSpec((Cout, 1), lambda b: (0, 0)),
                pl.BlockSpec((4 * N, 4 * N), lambda b: (0, 0)),
            ],
            out_specs=pl.BlockSpec((1, Cout, 4 * N), lambda b: (b, 0, 0)),
            compiler_params=pltpu.CompilerParams(
                dimension_semantics=("parallel",)),
        )(x2, s1col, s2t, w1s, b1s, wsk, bsk, w2f, b2, perm)

        return outp.reshape(B, Cout, 2 * H, 2 * W)


# speedup vs baseline: 23.9932x; 1.0678x over previous
import functools
import math

import numpy as np

import jax
import jax.numpy as jnp
from jax.experimental import pallas as pl
from jax.experimental.pallas import tpu as pltpu

_SQRT2 = math.sqrt(2.0)
_INV_SQRT2 = 1.0 / math.sqrt(2.0)


def _lrelu_scaled(v):
    return jnp.where(v >= 0.0, v, 0.2 * v) * _SQRT2


def _phase_row_ops():
    kh = np.array([1.0, 3.0, 3.0, 1.0], np.float64) / 4.0
    r = np.zeros((2, 3, 3), np.float64)
    for q in range(2):
        for ky in range(3):
            for a in range(3):
                idx = 2 * a + ky - 1 - q
                if 0 <= idx < 4:
                    r[q, ky, a] = kh[idx]
    return r.astype(np.float32)


def _phase_perm(h, w):
    n = h * w
    p = np.zeros((4 * n, 4 * n), np.float32)
    for qy in range(2):
        for qx in range(2):
            for y in range(h):
                for x in range(w):
                    src = (qy * 2 + qx) * n + y * w + x
                    dst = (2 * y + qy) * 2 * w + (2 * x + qx)
                    p[src, dst] = 1.0
    return p


def _fused_block_kernel(x_ref, s1_ref, s2t_ref, w1s_ref, b1s_ref,
                        wsk_ref, bsk_ref, w2f_ref, b2_ref, perm_ref, o_ref,
                        *, h, w, cin, cout, bb):
    for i in range(bb):
        _one_batch(i, x_ref, s1_ref, s2t_ref, w1s_ref, b1s_ref,
                   wsk_ref, bsk_ref, w2f_ref, b2_ref, perm_ref, o_ref,
                   h=h, w=w, cin=cin, cout=cout)


def _one_batch(i, x_ref, s1_ref, s2t_ref, w1s_ref, b1s_ref,
               wsk_ref, bsk_ref, w2f_ref, b2_ref, perm_ref, o_ref,
               *, h, w, cin, cout):
    n = h * w
    x = x_ref[i]
    pos = jax.lax.broadcasted_iota(jnp.int32, (1, n), 1)
    colv = pos % w
    rowv = pos // w

    def shift2d(v, dy, dx):
        s = dy * w + dx
        if s == 0:
            sh = v
        elif s > 0:
            sh = jnp.concatenate(
                [v[:, s:], jnp.zeros((v.shape[0], s), v.dtype)], axis=1)
        else:
            sh = jnp.concatenate(
                [jnp.zeros((v.shape[0], -s), v.dtype), v[:, :n + s]], axis=1)
        if dx != 0:
            sh = jnp.where((colv + dx >= 0) & (colv + dx < w), sh, 0.0)
        return sh

    xm = (x * s1_ref[i]).astype(jnp.bfloat16)
    xs = jnp.concatenate(
        [shift2d(xm, dy, dx) for dy in (-1, 0, 1) for dx in (-1, 0, 1)],
        axis=0)
    t = jnp.dot(w1s_ref[...], xs, preferred_element_type=jnp.float32,
                precision=jax.lax.Precision.DEFAULT)
    y1 = _lrelu_scaled(t + b1s_ref[...])
    ym = (y1 * s2t_ref[i]).astype(jnp.bfloat16)

    z = _lrelu_scaled(
        jnp.dot(wsk_ref[...], x, preferred_element_type=jnp.float32)
        + bsk_ref[...])
    zup = shift2d(z, -1, 0) + jnp.where(rowv == 0, z, 0.0)
    zdn = shift2d(z, 1, 0) + jnp.where(rowv == h - 1, z, 0.0)
    r0 = 0.75 * z + 0.25 * zup
    r1 = 0.75 * z + 0.25 * zdn

    def colmix(v):
        vl = shift2d(v, 0, -1) + jnp.where(colv == 0, v, 0.0)
        vr = shift2d(v, 0, 1) + jnp.where(colv == w - 1, v, 0.0)
        return 0.75 * v + 0.25 * vl, 0.75 * v + 0.25 * vr

    s00, s01 = colmix(r0)
    s10, s11 = colmix(r1)
    skip_all = jnp.concatenate([s00, s01, s10, s11], axis=1)

    cache = {}

    def plane_shifted(py, px, sy, sx):
        key = (py, px, sy, sx)
        if key not in cache:
            q = py * 2 + px
            cache[key] = shift2d(ym[q * cout:(q + 1) * cout], sy, sx)
        return cache[key]

    rowblocks = []
    for dy in (-1, 0, 1):
        for dx in (-1, 0, 1):
            qparts = []
            for qy in (0, 1):
                for qx in (0, 1):
                    py, sy = (qy + dy) % 2, (qy + dy) // 2
                    px, sx = (qx + dx) % 2, (qx + dx) // 2
                    qparts.append(plane_shifted(py, px, sy, sx))
            rowblocks.append(jnp.concatenate(qparts, axis=1))
    xs2 = jnp.concatenate(rowblocks, axis=0)

    res = _lrelu_scaled(
        jnp.dot(w2f_ref[...], xs2, preferred_element_type=jnp.float32,
                precision=jax.lax.Precision.DEFAULT)
        + b2_ref[...])
    outsum = ((skip_all + res) * _INV_SQRT2).astype(jnp.bfloat16)
    o_ref[i] = jnp.dot(outsum, perm_ref[...],
                       preferred_element_type=jnp.float32,
                       precision=jax.lax.Precision.DEFAULT)


def kernel(x, style, mod1_w, mod1_b, w1, act1_b,
           mod2_w, mod2_b, w2, act2_b, skip_w, skip_act_b):
    B, Cin, H, W = x.shape
    S = style.shape[1]
    Cout = w1.shape[1]
    N = H * W
    f32 = jnp.float32

    with jax.default_matmul_precision("highest"):
        def modulate(mw, mb):
            s = style @ (mw * (1.0 / math.sqrt(S))).T + mb
            return s * jax.lax.rsqrt(jnp.mean(s * s, axis=1, keepdims=True) + 1e-8)

        def demod_weight(w_raw, cin_):
            wd = w_raw[0] * (1.0 / math.sqrt(cin_ * 9))
            d = jax.lax.rsqrt(jnp.sum(wd * wd, axis=(1, 2, 3)) + 1e-8)
            return wd * d[:, None, None, None]

        s1 = modulate(mod1_w, mod1_b)
        s2 = modulate(mod2_w, mod2_b)
        w1d = demod_weight(w1, Cin)
        w2d = demod_weight(w2, Cout)

        rm = jnp.asarray(_phase_row_ops())
        w1s = jnp.einsum("ocyx,qya,rxb->qroabc", w1d, rm, rm,
                         ).reshape(4 * Cout, 9 * Cin).astype(jnp.bfloat16)
        w2f = jnp.transpose(w2d, (0, 2, 3, 1)).reshape(
            Cout, 9 * Cout).astype(jnp.bfloat16)

        s1col = s1[..., None]
        s2t = jnp.tile(s2, (1, 4))[..., None]
        b1s = jnp.tile(act1_b, 4)[:, None]
        wsk = skip_w * (1.0 / math.sqrt(Cin))
        bsk = skip_act_b[:, None]
        b2 = act2_b[:, None]

        x2 = x.reshape(B, Cin, N)
        perm = jnp.asarray(_phase_perm(H, W), jnp.bfloat16)
        BB = 2 if B % 2 == 0 else 1

        outp = pl.pallas_call(
            functools.partial(_fused_block_kernel, h=H, w=W, cin=Cin,
                              cout=Cout, bb=BB),
            out_shape=jax.ShapeDtypeStruct((B, Cout, 4 * N), f32),
            grid=(B // BB,),
            in_specs=[
                pl.BlockSpec((BB, Cin, N), lambda b: (b, 0, 0)),
                pl.BlockSpec((BB, Cin, 1), lambda b: (b, 0, 0)),
                pl.BlockSpec((BB, 4 * Cout, 1), lambda b: (b, 0, 0)),
                pl.BlockSpec((4 * Cout, 9 * Cin), lambda b: (0, 0)),
                pl.BlockSpec((4 * Cout, 1), lambda b: (0, 0)),
                pl.BlockSpec((Cout, Cin), lambda b: (0, 0)),
                pl.BlockSpec((Cout, 1), lambda b: (0, 0)),
                pl.BlockSpec((Cout, 9 * Cout), lambda b: (0, 0)),
                pl.BlockSpec((Cout, 1), lambda b: (0, 0)),
                pl.BlockSpec((4 * N, 4 * N), lambda b: (0, 0)),
            ],
            out_specs=pl.BlockSpec((BB, Cout, 4 * N), lambda b: (b, 0, 0)),
            compiler_params=pltpu.CompilerParams(
                dimension_semantics=("parallel",)),
        )(x2, s1col, s2t, w1s, b1s, wsk, bsk, w2f, b2, perm)

        return outp.reshape(B, Cout, 2 * H, 2 * W)


# speedup vs baseline: 25.1505x; 1.0482x over previous
import functools
import math

import numpy as np

import jax
import jax.numpy as jnp
from jax.experimental import pallas as pl
from jax.experimental.pallas import tpu as pltpu

_SQRT2 = math.sqrt(2.0)
_INV_SQRT2 = 1.0 / math.sqrt(2.0)


def _lrelu_scaled(v):
    return jnp.where(v >= 0.0, v, 0.2 * v) * _SQRT2


def _phase_row_ops():
    kh = np.array([1.0, 3.0, 3.0, 1.0], np.float64) / 4.0
    r = np.zeros((2, 3, 3), np.float64)
    for q in range(2):
        for ky in range(3):
            for a in range(3):
                idx = 2 * a + ky - 1 - q
                if 0 <= idx < 4:
                    r[q, ky, a] = kh[idx]
    return r.astype(np.float32)


def _phase_perm(h, w):
    n = h * w
    p = np.zeros((4 * n, 4 * n), np.float32)
    for qy in range(2):
        for qx in range(2):
            for y in range(h):
                for x in range(w):
                    src = (qy * 2 + qx) * n + y * w + x
                    dst = (2 * y + qy) * 2 * w + (2 * x + qx)
                    p[src, dst] = 1.0
    return p


def _fused_block_kernel(x_ref, s1_ref, s2t_ref, w1s_ref, b1s_ref,
                        wsk_ref, bsk_ref, w2f_ref, b2_ref, perm_ref, o_ref,
                        *, h, w, cin, cout, bb):
    for i in range(bb):
        _one_batch(i, x_ref, s1_ref, s2t_ref, w1s_ref, b1s_ref,
                   wsk_ref, bsk_ref, w2f_ref, b2_ref, perm_ref, o_ref,
                   h=h, w=w, cin=cin, cout=cout)


def _one_batch(i, x_ref, s1_ref, s2t_ref, w1s_ref, b1s_ref,
               wsk_ref, bsk_ref, w2f_ref, b2_ref, perm_ref, o_ref,
               *, h, w, cin, cout):
    n = h * w
    x = x_ref[i]
    pos = jax.lax.broadcasted_iota(jnp.int32, (1, n), 1)
    colv = pos % w
    rowv = pos // w

    def shift2d(v, dy, dx):
        s = dy * w + dx
        if s == 0:
            sh = v
        elif s > 0:
            sh = jnp.concatenate(
                [v[:, s:], jnp.zeros((v.shape[0], s), v.dtype)], axis=1)
        else:
            sh = jnp.concatenate(
                [jnp.zeros((v.shape[0], -s), v.dtype), v[:, :n + s]], axis=1)
        if dx != 0:
            sh = jnp.where((colv + dx >= 0) & (colv + dx < w), sh, 0.0)
        return sh

    xm = (x * s1_ref[i]).astype(jnp.bfloat16)
    xs = jnp.concatenate(
        [shift2d(xm, dy, dx) for dy in (-1, 0, 1) for dx in (-1, 0, 1)],
        axis=0)
    t = jnp.dot(w1s_ref[...], xs, preferred_element_type=jnp.float32,
                precision=jax.lax.Precision.DEFAULT)
    y1 = _lrelu_scaled(t + b1s_ref[...])
    ym = (y1 * s2t_ref[i]).astype(jnp.bfloat16)

    z = _lrelu_scaled(
        jnp.dot(wsk_ref[...], x, preferred_element_type=jnp.float32)
        + bsk_ref[...])
    zup = shift2d(z, -1, 0) + jnp.where(rowv == 0, z, 0.0)
    zdn = shift2d(z, 1, 0) + jnp.where(rowv == h - 1, z, 0.0)
    r0 = 0.75 * z + 0.25 * zup
    r1 = 0.75 * z + 0.25 * zdn

    def colmix(v):
        vl = shift2d(v, 0, -1) + jnp.where(colv == 0, v, 0.0)
        vr = shift2d(v, 0, 1) + jnp.where(colv == w - 1, v, 0.0)
        return 0.75 * v + 0.25 * vl, 0.75 * v + 0.25 * vr

    s00, s01 = colmix(r0)
    s10, s11 = colmix(r1)
    skip_all = jnp.concatenate([s00, s01, s10, s11], axis=1)

    cache = {}

    def plane_shifted(py, px, sy, sx):
        key = (py, px, sy, sx)
        if key not in cache:
            q = py * 2 + px
            cache[key] = shift2d(ym[q * cout:(q + 1) * cout], sy, sx)
        return cache[key]

    rowblocks = []
    for dy in (-1, 0, 1):
        for dx in (-1, 0, 1):
            qparts = []
            for qy in (0, 1):
                for qx in (0, 1):
                    py, sy = (qy + dy) % 2, (qy + dy) // 2
                    px, sx = (qx + dx) % 2, (qx + dx) // 2
                    qparts.append(plane_shifted(py, px, sy, sx))
            rowblocks.append(jnp.concatenate(qparts, axis=1))
    xs2 = jnp.concatenate(rowblocks, axis=0)

    res = _lrelu_scaled(
        jnp.dot(w2f_ref[...], xs2, preferred_element_type=jnp.float32,
                precision=jax.lax.Precision.DEFAULT)
        + b2_ref[...])
    outsum = ((skip_all + res) * _INV_SQRT2).astype(jnp.bfloat16)
    o_ref[i] = jnp.dot(outsum, perm_ref[...],
                       preferred_element_type=jnp.float32,
                       precision=jax.lax.Precision.DEFAULT)


def kernel(x, style, mod1_w, mod1_b, w1, act1_b,
           mod2_w, mod2_b, w2, act2_b, skip_w, skip_act_b):
    B, Cin, H, W = x.shape
    S = style.shape[1]
    Cout = w1.shape[1]
    N = H * W
    f32 = jnp.float32

    with jax.default_matmul_precision("highest"):
        def modulate(mw, mb):
            s = style @ (mw * (1.0 / math.sqrt(S))).T + mb
            return s * jax.lax.rsqrt(jnp.mean(s * s, axis=1, keepdims=True) + 1e-8)

        def demod_weight(w_raw, cin_):
            wd = w_raw[0] * (1.0 / math.sqrt(cin_ * 9))
            d = jax.lax.rsqrt(jnp.sum(wd * wd, axis=(1, 2, 3)) + 1e-8)
            return wd * d[:, None, None, None]

        s1 = modulate(mod1_w, mod1_b)
        s2 = modulate(mod2_w, mod2_b)
        w1d = demod_weight(w1, Cin)
        w2d = demod_weight(w2, Cout)

        rm = jnp.asarray(_phase_row_ops())
        w1s = jnp.einsum("ocyx,qya,rxb->qroabc", w1d, rm, rm,
                         ).reshape(4 * Cout, 9 * Cin).astype(jnp.bfloat16)
        w2f = jnp.transpose(w2d, (0, 2, 3, 1)).reshape(
            Cout, 9 * Cout).astype(jnp.bfloat16)

        s1col = s1[..., None]
        s2t = jnp.tile(s2, (1, 4))[..., None]
        b1s = jnp.tile(act1_b, 4)[:, None]
        wsk = skip_w * (1.0 / math.sqrt(Cin))
        bsk = skip_act_b[:, None]
        b2 = act2_b[:, None]

        x2 = x.reshape(B, Cin, N)
        perm = jnp.asarray(_phase_perm(H, W), jnp.bfloat16)
        BB = 4 if B % 4 == 0 else 1

        outp = pl.pallas_call(
            functools.partial(_fused_block_kernel, h=H, w=W, cin=Cin,
                              cout=Cout, bb=BB),
            out_shape=jax.ShapeDtypeStruct((B, Cout, 4 * N), f32),
            grid=(B // BB,),
            in_specs=[
                pl.BlockSpec((BB, Cin, N), lambda b: (b, 0, 0)),
                pl.BlockSpec((BB, Cin, 1), lambda b: (b, 0, 0)),
                pl.BlockSpec((BB, 4 * Cout, 1), lambda b: (b, 0, 0)),
                pl.BlockSpec((4 * Cout, 9 * Cin), lambda b: (0, 0)),
                pl.BlockSpec((4 * Cout, 1), lambda b: (0, 0)),
                pl.BlockSpec((Cout, Cin), lambda b: (0, 0)),
                pl.BlockSpec((Cout, 1), lambda b: (0, 0)),
                pl.BlockSpec((Cout, 9 * Cout), lambda b: (0, 0)),
                pl.BlockSpec((Cout, 1), lambda b: (0, 0)),
                pl.BlockSpec((4 * N, 4 * N), lambda b: (0, 0)),
            ],
            out_specs=pl.BlockSpec((BB, Cout, 4 * N), lambda b: (b, 0, 0)),
            compiler_params=pltpu.CompilerParams(
                dimension_semantics=("parallel",)),
        )(x2, s1col, s2t, w1s, b1s, wsk, bsk, w2f, b2, perm)

        return outp.reshape(B, Cout, 2 * H, 2 * W)


# speedup vs baseline: 25.6949x; 1.0216x over previous
import functools
import math

import numpy as np

import jax
import jax.numpy as jnp
from jax.experimental import pallas as pl
from jax.experimental.pallas import tpu as pltpu

_SQRT2 = math.sqrt(2.0)
_INV_SQRT2 = 1.0 / math.sqrt(2.0)


def _lrelu_scaled(v):
    return jnp.where(v >= 0.0, v, 0.2 * v) * _SQRT2


def _phase_row_ops():
    kh = np.array([1.0, 3.0, 3.0, 1.0], np.float64) / 4.0
    r = np.zeros((2, 3, 3), np.float64)
    for q in range(2):
        for ky in range(3):
            for a in range(3):
                idx = 2 * a + ky - 1 - q
                if 0 <= idx < 4:
                    r[q, ky, a] = kh[idx]
    return r.astype(np.float32)


def _phase_perm(h, w):
    n = h * w
    p = np.zeros((4 * n, 4 * n), np.float32)
    for qy in range(2):
        for qx in range(2):
            for y in range(h):
                for x in range(w):
                    src = (qy * 2 + qx) * n + y * w + x
                    dst = (2 * y + qy) * 2 * w + (2 * x + qx)
                    p[src, dst] = 1.0
    return p


def _fused_block_kernel(x_ref, s1_ref, s2t_ref, w1s_ref, b1s_ref,
                        wsk_ref, bsk_ref, w2f_ref, b2_ref, perm_ref, o_ref,
                        *, h, w, cin, cout, bb):
    for i in range(bb):
        _one_batch(i, x_ref, s1_ref, s2t_ref, w1s_ref, b1s_ref,
                   wsk_ref, bsk_ref, w2f_ref, b2_ref, perm_ref, o_ref,
                   h=h, w=w, cin=cin, cout=cout)


def _one_batch(i, x_ref, s1_ref, s2t_ref, w1s_ref, b1s_ref,
               wsk_ref, bsk_ref, w2f_ref, b2_ref, perm_ref, o_ref,
               *, h, w, cin, cout):
    n = h * w
    x = x_ref[i]
    pos = jax.lax.broadcasted_iota(jnp.int32, (1, n), 1)
    colv = pos % w
    rowv = pos // w

    def shift2d(v, dy, dx):
        s = dy * w + dx
        if s == 0:
            sh = v
        elif s > 0:
            sh = jnp.concatenate(
                [v[:, s:], jnp.zeros((v.shape[0], s), v.dtype)], axis=1)
        else:
            sh = jnp.concatenate(
                [jnp.zeros((v.shape[0], -s), v.dtype), v[:, :n + s]], axis=1)
        if dx != 0:
            sh = jnp.where((colv + dx >= 0) & (colv + dx < w), sh, 0.0)
        return sh

    xm = (x * s1_ref[i]).astype(jnp.bfloat16)
    xs = jnp.concatenate(
        [shift2d(xm, dy, dx) for dy in (-1, 0, 1) for dx in (-1, 0, 1)],
        axis=0)
    t = jnp.dot(w1s_ref[...], xs, preferred_element_type=jnp.float32,
                precision=jax.lax.Precision.DEFAULT)
    y1 = _lrelu_scaled(t + b1s_ref[...])
    ym = (y1 * s2t_ref[i]).astype(jnp.bfloat16)

    z = _lrelu_scaled(
        jnp.dot(wsk_ref[...], x, preferred_element_type=jnp.float32)
        + bsk_ref[...])
    zup = shift2d(z, -1, 0) + jnp.where(rowv == 0, z, 0.0)
    zdn = shift2d(z, 1, 0) + jnp.where(rowv == h - 1, z, 0.0)
    r0 = 0.75 * z + 0.25 * zup
    r1 = 0.75 * z + 0.25 * zdn

    def colmix(v):
        vl = shift2d(v, 0, -1) + jnp.where(colv == 0, v, 0.0)
        vr = shift2d(v, 0, 1) + jnp.where(colv == w - 1, v, 0.0)
        return 0.75 * v + 0.25 * vl, 0.75 * v + 0.25 * vr

    s00, s01 = colmix(r0)
    s10, s11 = colmix(r1)
    skip_all = jnp.concatenate([s00, s01, s10, s11], axis=1)

    cache = {}

    def plane_shifted(py, px, sy, sx):
        key = (py, px, sy, sx)
        if key not in cache:
            q = py * 2 + px
            cache[key] = shift2d(ym[q * cout:(q + 1) * cout], sy, sx)
        return cache[key]

    rowblocks = []
    for dy in (-1, 0, 1):
        for dx in (-1, 0, 1):
            qparts = []
            for qy in (0, 1):
                for qx in (0, 1):
                    py, sy = (qy + dy) % 2, (qy + dy) // 2
                    px, sx = (qx + dx) % 2, (qx + dx) // 2
                    qparts.append(plane_shifted(py, px, sy, sx))
            rowblocks.append(jnp.concatenate(qparts, axis=1))
    xs2 = jnp.concatenate(rowblocks, axis=0)

    res = _lrelu_scaled(
        jnp.dot(w2f_ref[...], xs2, preferred_element_type=jnp.float32,
                precision=jax.lax.Precision.DEFAULT)
        + b2_ref[...])
    outsum = ((skip_all + res) * _INV_SQRT2).astype(jnp.bfloat16)
    o_ref[i] = jnp.dot(outsum, perm_ref[...],
                       preferred_element_type=jnp.float32,
                       precision=jax.lax.Precision.DEFAULT)


def kernel(x, style, mod1_w, mod1_b, w1, act1_b,
           mod2_w, mod2_b, w2, act2_b, skip_w, skip_act_b):
    B, Cin, H, W = x.shape
    S = style.shape[1]
    Cout = w1.shape[1]
    N = H * W
    f32 = jnp.float32

    with jax.default_matmul_precision("highest"):
        def modulate(mw, mb):
            s = style @ (mw * (1.0 / math.sqrt(S))).T + mb
            return s * jax.lax.rsqrt(jnp.mean(s * s, axis=1, keepdims=True) + 1e-8)

        def demod_weight(w_raw, cin_):
            wd = w_raw[0] * (1.0 / math.sqrt(cin_ * 9))
            d = jax.lax.rsqrt(jnp.sum(wd * wd, axis=(1, 2, 3)) + 1e-8)
            return wd * d[:, None, None, None]

        s1 = modulate(mod1_w, mod1_b)
        s2 = modulate(mod2_w, mod2_b)
        w1d = demod_weight(w1, Cin)
        w2d = demod_weight(w2, Cout)

        rm = jnp.asarray(_phase_row_ops())
        w1s = jnp.einsum("ocyx,qya,rxb->qroabc", w1d, rm, rm,
                         ).reshape(4 * Cout, 9 * Cin).astype(jnp.bfloat16)
        w2f = jnp.transpose(w2d, (0, 2, 3, 1)).reshape(
            Cout, 9 * Cout).astype(jnp.bfloat16)

        s1col = s1[..., None]
        s2t = jnp.tile(s2, (1, 4))[..., None]
        b1s = jnp.tile(act1_b, 4)[:, None]
        wsk = skip_w * (1.0 / math.sqrt(Cin))
        bsk = skip_act_b[:, None]
        b2 = act2_b[:, None]

        x2 = x.reshape(B, Cin, N)
        perm = jnp.asarray(_phase_perm(H, W), jnp.bfloat16)
        BB = 8 if B % 8 == 0 else 1

        outp = pl.pallas_call(
            functools.partial(_fused_block_kernel, h=H, w=W, cin=Cin,
                              cout=Cout, bb=BB),
            out_shape=jax.ShapeDtypeStruct((B, Cout, 4 * N), f32),
            grid=(B // BB,),
            in_specs=[
                pl.BlockSpec((BB, Cin, N), lambda b: (b, 0, 0)),
                pl.BlockSpec((BB, Cin, 1), lambda b: (b, 0, 0)),
                pl.BlockSpec((BB, 4 * Cout, 1), lambda b: (b, 0, 0)),
                pl.BlockSpec((4 * Cout, 9 * Cin), lambda b: (0, 0)),
                pl.BlockSpec((4 * Cout, 1), lambda b: (0, 0)),
                pl.BlockSpec((Cout, Cin), lambda b: (0, 0)),
                pl.BlockSpec((Cout, 1), lambda b: (0, 0)),
                pl.BlockSpec((Cout, 9 * Cout), lambda b: (0, 0)),
                pl.BlockSpec((Cout, 1), lambda b: (0, 0)),
                pl.BlockSpec((4 * N, 4 * N), lambda b: (0, 0)),
            ],
            out_specs=pl.BlockSpec((BB, Cout, 4 * N), lambda b: (b, 0, 0)),
            compiler_params=pltpu.CompilerParams(
                dimension_semantics=("parallel",)),
        )(x2, s1col, s2t, w1s, b1s, wsk, bsk, w2f, b2, perm)

        return outp.reshape(B, Cout, 2 * H, 2 * W)
